# Initial kernel scaffold; baseline (speedup 1.0000x reference)
import functools

import jax
import jax.numpy as jnp
from jax import lax
from jax.experimental import pallas as pl
from jax.experimental.pallas import tpu as pltpu
from jax.experimental.pallas import tpu_sc as plsc

NC = 2
NS = 16
NW = NC * NS
C = 128


def _ceil_to(x, m):
    return (x + m - 1) // m * m


@functools.lru_cache(maxsize=None)
def _make_stage1(N, D, NP, EPW):
    nchunks = EPW // C
    RPT = NP // NS
    NZ = RPT // C
    mesh = plsc.VectorSubcoreMesh(core_axis_name="c", subcore_axis_name="s")

    @functools.partial(
        pl.kernel,
        out_type=[
            jax.ShapeDtypeStruct((NC, NP, D), jnp.float32),
            jax.ShapeDtypeStruct((NC, NP, 16), jnp.float32),
            jax.ShapeDtypeStruct((NC, NP, 16), jnp.float32),
        ],
        mesh=mesh,
        scratch_types=[
            pltpu.VMEM((C,), jnp.int32),
            pltpu.VMEM((C,), jnp.int32),
            pltpu.VMEM((C,), jnp.int32),
            pltpu.VMEM((C, D), jnp.float32),
            pltpu.VMEM((C, 16), jnp.float32),
            pltpu.VMEM_SHARED((NP, D), jnp.float32),
            pltpu.VMEM_SHARED((NP, 16), jnp.float32),
            pltpu.VMEM_SHARED((NP, 16), jnp.float32),
            pltpu.SemaphoreType.DMA,
        ],
    )
    def stage1(x_hbm, srcg_hbm, dsts_hbm, srcs_hbm,
               agg_out, deg_out, cnt_out,
               idx_g, idx_d, idx_s, rows, onesv,
               agg_sh, deg_sh, cnt_sh, sem):
        cid = lax.axis_index("c")
        sid = lax.axis_index("s")
        wid = sid * NC + cid
        zero16 = jnp.zeros((16,), jnp.float32)
        one16 = jnp.ones((16,), jnp.float32)

        def _zrow(r, _):
            for k in range(D // 16):
                rows[r, pl.ds(k * 16, 16)] = zero16
            onesv[r, pl.ds(0, 16)] = zero16
            return 0

        lax.fori_loop(0, C, _zrow, 0)

        r0 = sid * RPT
        for t in range(NZ):
            pltpu.sync_copy(rows, agg_sh.at[pl.ds(r0 + t * C, C)])
            pltpu.sync_copy(onesv, deg_sh.at[pl.ds(r0 + t * C, C)])
            pltpu.sync_copy(onesv, cnt_sh.at[pl.ds(r0 + t * C, C)])

        def _orow(r, _):
            onesv[r, pl.ds(0, 16)] = one16
            return 0

        lax.fori_loop(0, C, _orow, 0)
        plsc.subcore_barrier()

        def _chunk(j, _):
            base = wid * EPW + j * C
            pltpu.sync_copy(srcg_hbm.at[pl.ds(base, C)], idx_g)
            pltpu.sync_copy(dsts_hbm.at[pl.ds(base, C)], idx_d)
            pltpu.sync_copy(srcs_hbm.at[pl.ds(base, C)], idx_s)
            pltpu.async_copy(x_hbm.at[idx_g], rows, sem).wait()
            pltpu.sync_copy(rows, agg_sh.at[idx_d], add=True)
            pltpu.sync_copy(onesv, deg_sh.at[idx_d], add=True)
            pltpu.sync_copy(onesv, cnt_sh.at[idx_s], add=True)
            return 0

        lax.fori_loop(0, nchunks, _chunk, 0)
        plsc.subcore_barrier()

        pltpu.sync_copy(agg_sh.at[pl.ds(r0, RPT)], agg_out.at[cid, pl.ds(r0, RPT)])
        pltpu.sync_copy(deg_sh.at[pl.ds(r0, RPT)], deg_out.at[cid, pl.ds(r0, RPT)])
        pltpu.sync_copy(cnt_sh.at[pl.ds(r0, RPT)], cnt_out.at[cid, pl.ds(r0, RPT)])

    return stage1


@functools.lru_cache(maxsize=None)
def _make_stage2(N, D, NP, EPW):
    nchunks = EPW // C
    RPT = NP // NS
    NZ = RPT // C
    mesh = plsc.VectorSubcoreMesh(core_axis_name="c", subcore_axis_name="s")

    @functools.partial(
        pl.kernel,
        out_type=jax.ShapeDtypeStruct((NC, NP, D), jnp.float32),
        mesh=mesh,
        scratch_types=[
            pltpu.VMEM((C,), jnp.int32),
            pltpu.VMEM((C,), jnp.int32),
            pltpu.VMEM((C,), jnp.int32),
            pltpu.VMEM((C, D), jnp.float32),
            pltpu.VMEM((C, D), jnp.float32),
            pltpu.VMEM_SHARED((NP, D), jnp.float32),
            pltpu.SemaphoreType.DMA,
            pltpu.SemaphoreType.DMA,
        ],
    )
    def stage2(h_hbm, srcg_hbm, dstg_hbm, srcs_hbm,
               s_out,
               idx_a, idx_b, idx_s, rows_a, rows_b,
               s_sh, sem_a, sem_b):
        cid = lax.axis_index("c")
        sid = lax.axis_index("s")
        wid = sid * NC + cid
        zero16 = jnp.zeros((16,), jnp.float32)

        def _zrow(r, _):
            for k in range(D // 16):
                rows_a[r, pl.ds(k * 16, 16)] = zero16
            return 0

        lax.fori_loop(0, C, _zrow, 0)

        r0 = sid * RPT
        for t in range(NZ):
            pltpu.sync_copy(rows_a, s_sh.at[pl.ds(r0 + t * C, C)])
        plsc.subcore_barrier()

        def _chunk(j, _):
            base = wid * EPW + j * C
            pltpu.sync_copy(srcg_hbm.at[pl.ds(base, C)], idx_a)
            pltpu.sync_copy(dstg_hbm.at[pl.ds(base, C)], idx_b)
            pltpu.sync_copy(srcs_hbm.at[pl.ds(base, C)], idx_s)
            ca = pltpu.async_copy(h_hbm.at[idx_a], rows_a, sem_a)
            cb = pltpu.async_copy(h_hbm.at[idx_b], rows_b, sem_b)
            ca.wait()
            cb.wait()

            def _erow(r, _):
                for k in range(D // 16):
                    a = rows_a[r, pl.ds(k * 16, 16)]
                    bb = rows_b[r, pl.ds(k * 16, 16)]
                    d = a - bb
                    rows_a[r, pl.ds(k * 16, 16)] = d * d
                return 0

            lax.fori_loop(0, C, _erow, 0)
            pltpu.sync_copy(rows_a, s_sh.at[idx_s], add=True)
            return 0

        lax.fori_loop(0, nchunks, _chunk, 0)
        plsc.subcore_barrier()
        pltpu.sync_copy(s_sh.at[pl.ds(r0, RPT)], s_out.at[cid, pl.ds(r0, RPT)])

    return stage2


@functools.lru_cache(maxsize=None)
def _make_conv(N, D, BN):
    def body(x_ref, agg_ref, deg_ref, ws_ref, wn_ref, b_ref, h_ref):
        deg = deg_ref[0, :, 0:1] + deg_ref[1, :, 0:1]
        mean = (agg_ref[0] + agg_ref[1]) / jnp.maximum(deg, 1.0)
        h = jnp.dot(x_ref[...], ws_ref[...], preferred_element_type=jnp.float32)
        h = h + jnp.dot(mean, wn_ref[...], preferred_element_type=jnp.float32)
        h = h + b_ref[...]
        h_ref[...] = jnp.maximum(h, 0.0)

    return pl.pallas_call(
        body,
        grid=(N // BN,),
        in_specs=[
            pl.BlockSpec((BN, D), lambda i: (i, 0)),
            pl.BlockSpec((NC, BN, D), lambda i: (0, i, 0)),
            pl.BlockSpec((NC, BN, 16), lambda i: (0, i, 0)),
            pl.BlockSpec((D, D), lambda i: (0, 0)),
            pl.BlockSpec((D, D), lambda i: (0, 0)),
            pl.BlockSpec((1, D), lambda i: (0, 0)),
        ],
        out_specs=pl.BlockSpec((BN, D), lambda i: (i, 0)),
        out_shape=jax.ShapeDtypeStruct((N, D), jnp.float32),
    )


@functools.lru_cache(maxsize=None)
def _make_final(N, D, BN):
    def body(s_ref, cnt_ref, gg_ref):
        cnt = cnt_ref[0, :, 0:1] + cnt_ref[1, :, 0:1]
        gg_ref[...] = jnp.tanh((s_ref[0] + s_ref[1]) / jnp.maximum(cnt, 1.0))

    return pl.pallas_call(
        body,
        grid=(N // BN,),
        in_specs=[
            pl.BlockSpec((NC, BN, D), lambda i: (0, i, 0)),
            pl.BlockSpec((NC, BN, 16), lambda i: (0, i, 0)),
        ],
        out_specs=pl.BlockSpec((BN, D), lambda i: (i, 0)),
        out_shape=jax.ShapeDtypeStruct((N, D), jnp.float32),
    )


def kernel(X, edge_index, W_self, W_neigh, b):
    N, D = X.shape
    E = edge_index.shape[1]
    NP = _ceil_to(N + 1, NS * C)
    E_pad = _ceil_to(E, NW * C)
    EPW = E_pad // NW

    src = edge_index[0]
    dst = edge_index[1]
    pad = E_pad - E
    zpad = jnp.zeros((pad,), jnp.int32)
    gpad = jnp.full((pad,), N, jnp.int32)
    src_g = jnp.concatenate([src, zpad])
    dst_g = jnp.concatenate([dst, zpad])
    src_s = jnp.concatenate([src, gpad])
    dst_s = jnp.concatenate([dst, gpad])

    agg2, deg2, cnt2 = _make_stage1(N, D, NP, EPW)(X, src_g, dst_s, src_s)
    h = _make_conv(N, D, 400)(X, agg2, deg2, W_self, W_neigh, b.reshape(1, D))
    s2 = _make_stage2(N, D, NP, EPW)(h, src_g, dst_g, src_s)
    gg = _make_final(N, D, 400)(s2, cnt2)
    return gg


# speedup vs baseline: 3.5850x; 3.5850x over previous
import functools

import jax
import jax.numpy as jnp
from jax import lax
from jax.experimental import pallas as pl
from jax.experimental.pallas import tpu as pltpu
from jax.experimental.pallas import tpu_sc as plsc

NC = 2
NS = 16
NW = NC * NS
C = 128


def _ceil_to(x, m):
    return (x + m - 1) // m * m


@functools.lru_cache(maxsize=None)
def _make_stage1(N, D, NP, EPW, EPT):
    nchunks = EPW // C
    hchunks = EPT // C
    RPT = NP // NS
    NZ = RPT // C
    mesh = plsc.VectorSubcoreMesh(core_axis_name="c", subcore_axis_name="s")

    @functools.partial(
        pl.kernel,
        out_type=[
            jax.ShapeDtypeStruct((NC * NP, D), jnp.float32),
            jax.ShapeDtypeStruct((NC * NP,), jnp.float32),
        ],
        mesh=mesh,
        scratch_types=[
            pltpu.VMEM((C,), jnp.int32),
            pltpu.VMEM((C,), jnp.int32),
            pltpu.VMEM((C, D), jnp.float32),
            pltpu.VMEM((C,), jnp.float32),
            pltpu.VMEM((RPT,), jnp.float32),
            pltpu.VMEM_SHARED((NP, D), jnp.float32),
            pltpu.VMEM_SHARED((NP,), jnp.float32),
            pltpu.SemaphoreType.DMA,
        ],
    )
    def stage1(x_hbm, srcg_hbm, dsts_hbm, histidx_hbm,
               agg_out, hist_out,
               idx_g, idx_d, rows, ones1, hbounce,
               agg_sh, hist_sh, sem):
        cid = lax.axis_index("c")
        sid = lax.axis_index("s")
        wid = sid * NC + cid
        zero16 = jnp.zeros((16,), jnp.float32)
        one16 = jnp.ones((16,), jnp.float32)

        def _zrow(r, _):
            for k in range(D // 16):
                rows[r, pl.ds(k * 16, 16)] = zero16
            return 0

        lax.fori_loop(0, C, _zrow, 0)
        for k in range(C // 16):
            ones1[pl.ds(k * 16, 16)] = one16

        def _zh(r, _):
            hbounce[pl.ds(r * 16, 16)] = zero16
            return 0

        lax.fori_loop(0, RPT // 16, _zh, 0)

        r0 = sid * RPT
        for t in range(NZ):
            pltpu.sync_copy(rows, agg_sh.at[pl.ds(r0 + t * C, C)])
        pltpu.sync_copy(hbounce, hist_sh.at[pl.ds(r0, RPT)])
        plsc.subcore_barrier()

        def _chunk(j, _):
            base = wid * EPW + j * C
            pltpu.sync_copy(srcg_hbm.at[pl.ds(base, C)], idx_g)
            pltpu.sync_copy(dsts_hbm.at[pl.ds(base, C)], idx_d)
            pltpu.async_copy(x_hbm.at[idx_g], rows, sem).wait()
            pltpu.sync_copy(rows, agg_sh.at[idx_d], add=True)
            return 0

        lax.fori_loop(0, nchunks, _chunk, 0)

        def _hchunk(j, _):
            base = cid * (EPT * NS) + sid * EPT + j * C
            pltpu.sync_copy(histidx_hbm.at[pl.ds(base, C)], idx_d)
            pltpu.sync_copy(ones1, hist_sh.at[idx_d], add=True)
            return 0

        lax.fori_loop(0, hchunks, _hchunk, 0)
        plsc.subcore_barrier()

        for t in range(NZ):
            pltpu.sync_copy(agg_sh.at[pl.ds(r0 + t * C, C)], rows)
            pltpu.sync_copy(rows, agg_out.at[pl.ds(cid * NP + r0 + t * C, C)])
        pltpu.sync_copy(hist_sh.at[pl.ds(r0, RPT)], hbounce)
        pltpu.sync_copy(hbounce, hist_out.at[pl.ds(cid * NP + r0, RPT)])

    return stage1


@functools.lru_cache(maxsize=None)
def _make_stage2(N, D, NP, EPW):
    nchunks = EPW // C
    RPT = NP // NS
    NZ = RPT // C
    mesh = plsc.VectorSubcoreMesh(core_axis_name="c", subcore_axis_name="s")

    @functools.partial(
        pl.kernel,
        out_type=jax.ShapeDtypeStruct((NC * NP, D), jnp.float32),
        mesh=mesh,
        scratch_types=[
            pltpu.VMEM((C,), jnp.int32),
            pltpu.VMEM((C,), jnp.int32),
            pltpu.VMEM((C,), jnp.int32),
            pltpu.VMEM((C, D), jnp.float32),
            pltpu.VMEM((C, D), jnp.float32),
            pltpu.VMEM_SHARED((NP, D), jnp.float32),
            pltpu.SemaphoreType.DMA,
            pltpu.SemaphoreType.DMA,
        ],
    )
    def stage2(h_hbm, srcg_hbm, dstg_hbm, srcs_hbm,
               s_out,
               idx_a, idx_b, idx_s, rows_a, rows_b,
               s_sh, sem_a, sem_b):
        cid = lax.axis_index("c")
        sid = lax.axis_index("s")
        wid = sid * NC + cid
        zero16 = jnp.zeros((16,), jnp.float32)

        def _zrow(r, _):
            for k in range(D // 16):
                rows_a[r, pl.ds(k * 16, 16)] = zero16
            return 0

        lax.fori_loop(0, C, _zrow, 0)
        r0 = sid * RPT
        for t in range(NZ):
            pltpu.sync_copy(rows_a, s_sh.at[pl.ds(r0 + t * C, C)])
        plsc.subcore_barrier()

        def _chunk(j, _):
            base = wid * EPW + j * C
            pltpu.sync_copy(srcg_hbm.at[pl.ds(base, C)], idx_a)
            pltpu.sync_copy(dstg_hbm.at[pl.ds(base, C)], idx_b)
            pltpu.sync_copy(srcs_hbm.at[pl.ds(base, C)], idx_s)
            ca = pltpu.async_copy(h_hbm.at[idx_a], rows_a, sem_a)
            cb = pltpu.async_copy(h_hbm.at[idx_b], rows_b, sem_b)
            ca.wait()
            cb.wait()

            def _erow(r, _):
                for k in range(D // 16):
                    a = rows_a[r, pl.ds(k * 16, 16)]
                    bb = rows_b[r, pl.ds(k * 16, 16)]
                    d = a - bb
                    rows_a[r, pl.ds(k * 16, 16)] = d * d
                return 0

            lax.fori_loop(0, C, _erow, 0)
            pltpu.sync_copy(rows_a, s_sh.at[idx_s], add=True)
            return 0

        lax.fori_loop(0, nchunks, _chunk, 0)
        plsc.subcore_barrier()
        for t in range(NZ):
            pltpu.sync_copy(s_sh.at[pl.ds(r0 + t * C, C)], rows_a)
            pltpu.sync_copy(rows_a, s_out.at[pl.ds(cid * NP + r0 + t * C, C)])

    return stage2


@functools.lru_cache(maxsize=None)
def _make_conv(N, D, BN):
    def body(x_ref, agg_ref, deg_ref, ws_ref, wn_ref, b_ref, h_ref):
        deg = jnp.maximum(deg_ref[...], 1.0)
        mean = (agg_ref[0] + agg_ref[1]) / deg
        h = jnp.dot(x_ref[...], ws_ref[...], preferred_element_type=jnp.float32)
        h = h + jnp.dot(mean, wn_ref[...], preferred_element_type=jnp.float32)
        h = h + b_ref[...]
        h_ref[...] = jnp.maximum(h, 0.0)

    return pl.pallas_call(
        body,
        grid=(N // BN,),
        in_specs=[
            pl.BlockSpec((BN, D), lambda i: (i, 0)),
            pl.BlockSpec((NC, BN, D), lambda i: (0, i, 0)),
            pl.BlockSpec((BN, 1), lambda i: (i, 0)),
            pl.BlockSpec((D, D), lambda i: (0, 0)),
            pl.BlockSpec((D, D), lambda i: (0, 0)),
            pl.BlockSpec((1, D), lambda i: (0, 0)),
        ],
        out_specs=pl.BlockSpec((BN, D), lambda i: (i, 0)),
        out_shape=jax.ShapeDtypeStruct((N, D), jnp.float32),
    )


@functools.lru_cache(maxsize=None)
def _make_final(N, D, BN):
    def body(s_ref, cnt_ref, gg_ref):
        cnt = jnp.maximum(cnt_ref[...], 1.0)
        gg_ref[...] = jnp.tanh((s_ref[0] + s_ref[1]) / cnt)

    return pl.pallas_call(
        body,
        grid=(N // BN,),
        in_specs=[
            pl.BlockSpec((NC, BN, D), lambda i: (0, i, 0)),
            pl.BlockSpec((BN, 1), lambda i: (i, 0)),
        ],
        out_specs=pl.BlockSpec((BN, D), lambda i: (i, 0)),
        out_shape=jax.ShapeDtypeStruct((N, D), jnp.float32),
    )


def kernel(X, edge_index, W_self, W_neigh, b):
    N, D = X.shape
    E = edge_index.shape[1]
    NP = _ceil_to(N + 1, NS * C)
    E_pad = _ceil_to(E, NW * C)
    EPW = E_pad // NW
    EPT = E_pad // NS

    src = edge_index[0]
    dst = edge_index[1]
    pad = E_pad - E
    zpad = jnp.zeros((pad,), jnp.int32)
    gpad = jnp.full((pad,), N, jnp.int32)
    src_g = jnp.concatenate([src, zpad])
    dst_g = jnp.concatenate([dst, zpad])
    src_s = jnp.concatenate([src, gpad])
    dst_s = jnp.concatenate([dst, gpad])
    hist_idx = jnp.concatenate([dst_s, src_s])

    agg2, hist2 = _make_stage1(N, D, NP, EPW, EPT)(X, src_g, dst_s, hist_idx)
    agg2 = agg2.reshape(NC, NP, D)
    hist2 = hist2.reshape(NC, NP)
    deg_col = hist2[0, :N, None]
    cnt_col = hist2[1, :N, None]
    h = _make_conv(N, D, 400)(X, agg2, deg_col, W_self, W_neigh, b.reshape(1, D))
    s2 = _make_stage2(N, D, NP, EPW)(h, src_g, dst_g, src_s)
    s2 = s2.reshape(NC, NP, D)
    gg = _make_final(N, D, 400)(s2, cnt_col)
    return gg
